# Initial kernel scaffold; baseline (speedup 1.0000x reference)
import jax
import jax.numpy as jnp
from jax.experimental import pallas as pl


def kernel(x, edge_index, W1, b1, W2, b2, alpha, label_init, train_mask, hard_one_hot):
    raise NotImplementedError("write your pallas kernel here")


# speedup vs baseline: 10.4083x; 10.4083x over previous
import functools

import jax
import jax.numpy as jnp
from jax import lax
from jax.experimental import pallas as pl
from jax.experimental.pallas import tpu as pltpu
from jax.experimental.pallas import tpu_sc as plsc

N = 10000
E = 320000
D = 128
H = 256
C = 40
STEPS = 10

NP = 10240
CP = 48
K = 128
NW = 32
NCH = 80
EPW = NCH * K
EPAD = NW * EPW
ROWS_PT = NP // 16
PH = NP // NW
PH2 = PH // 2

_mesh = plsc.VectorSubcoreMesh(core_axis_name="c", subcore_axis_name="s")
_f32 = jnp.float32


def _zero_buf(ref, nrows):
    def body(r, carry):
        for cc in (0, 16, 32):
            ref[r, pl.ds(cc, 16)] = jnp.zeros((16,), _f32)
        return carry
    lax.fori_loop(0, nrows, body, 0)


def _zero_acc_slice(zbuf, acc, s):
    def body(k, carry):
        pltpu.sync_copy(zbuf, acc.at[pl.ds(s * ROWS_PT + k * K, K)])
        return carry
    lax.fori_loop(0, ROWS_PT // K, body, 0)


def _gather_scatter(qsrc, rows_hbm, cols_hbm, idxr, idxc, buf, acc, w):
    pltpu.sync_copy(rows_hbm.at[w], idxr)
    pltpu.sync_copy(cols_hbm.at[w], idxc)
    def body(j, carry):
        pltpu.sync_copy(qsrc.at[idxr.at[j]], buf)
        pltpu.sync_copy(buf, acc.at[idxc.at[j]], add=True)
        return carry
    lax.fori_loop(0, NCH, body, 0)


@functools.partial(
    pl.kernel,
    mesh=_mesh,
    compiler_params=pltpu.CompilerParams(use_tc_tiling_on_sc=False),
    out_type=jax.ShapeDtypeStruct((2, NP), _f32),
    scratch_types=[
        pltpu.VMEM((NCH, K), jnp.int32),
        pltpu.VMEM((K,), _f32),
        pltpu.VMEM((ROWS_PT,), _f32),
        pltpu.VMEM_SHARED((NP,), _f32),
    ],
)
def _deg_kernel(rows_hbm, out_hbm, idx_v, ones_v, z_v, acc):
    c = lax.axis_index("c")
    s = lax.axis_index("s")
    w = c * 16 + s
    def zb(i, carry):
        z_v[pl.ds(i * 16, 16)] = jnp.zeros((16,), _f32)
        return carry
    lax.fori_loop(0, ROWS_PT // 16, zb, 0)
    def ob(i, carry):
        ones_v[pl.ds(i * 16, 16)] = jnp.ones((16,), _f32)
        return carry
    lax.fori_loop(0, K // 16, ob, 0)
    pltpu.sync_copy(z_v, acc.at[pl.ds(s * ROWS_PT, ROWS_PT)])
    pltpu.sync_copy(rows_hbm.at[w], idx_v)
    plsc.subcore_barrier()
    def eb(j, carry):
        pltpu.sync_copy(ones_v, acc.at[idx_v.at[j]], add=True)
        return carry
    lax.fori_loop(0, NCH, eb, 0)
    plsc.subcore_barrier()
    pltpu.sync_copy(acc.at[pl.ds(s * ROWS_PT, ROWS_PT)],
                    out_hbm.at[c, pl.ds(s * ROWS_PT, ROWS_PT)])


@functools.partial(
    pl.kernel,
    mesh=_mesh,
    compiler_params=pltpu.CompilerParams(use_tc_tiling_on_sc=False),
    out_type=jax.ShapeDtypeStruct((2, NP, CP), _f32),
    scratch_types=[
        pltpu.VMEM((NCH, K), jnp.int32),
        pltpu.VMEM((NCH, K), jnp.int32),
        pltpu.VMEM((K, CP), _f32),
        pltpu.VMEM((K, CP), _f32),
        pltpu.VMEM_SHARED((NP, CP), _f32),
    ],
)
def _step_first(q0_hbm, rows_hbm, cols_hbm, p_out, idxr, idxc, buf, zbuf, acc):
    c = lax.axis_index("c")
    s = lax.axis_index("s")
    w = c * 16 + s
    _zero_buf(zbuf, K)
    _zero_acc_slice(zbuf, acc, s)
    plsc.subcore_barrier()
    _gather_scatter(q0_hbm, rows_hbm, cols_hbm, idxr, idxc, buf, acc, w)
    plsc.subcore_barrier()
    pltpu.sync_copy(acc.at[pl.ds(s * ROWS_PT, ROWS_PT)],
                    p_out.at[c, pl.ds(s * ROWS_PT, ROWS_PT)])


@functools.partial(
    pl.kernel,
    mesh=_mesh,
    compiler_params=pltpu.CompilerParams(use_tc_tiling_on_sc=False),
    out_type=(jax.ShapeDtypeStruct((2, NP, CP), _f32),
              jax.ShapeDtypeStruct((NP, CP), _f32)),
    scratch_types=[
        pltpu.VMEM((NCH, K), jnp.int32),
        pltpu.VMEM((NCH, K), jnp.int32),
        pltpu.VMEM((K, CP), _f32),
        pltpu.VMEM((K, CP), _f32),
        pltpu.VMEM((PH2, CP), _f32),
        pltpu.VMEM((PH2, CP), _f32),
        pltpu.VMEM((PH2, CP), _f32),
        pltpu.VMEM((PH2, CP), _f32),
        pltpu.VMEM((PH2, CP), _f32),
        pltpu.VMEM_SHARED((NP, CP), _f32),
        pltpu.SemaphoreType.REGULAR,
    ],
)
def _step_rest(pprev, qprev, qh, d2, rows_hbm, cols_hbm, p_out, q_out,
               idxr, idxc, buf, zbuf, ca, cb, cq, ch, cd, acc, bsem):
    c = lax.axis_index("c")
    s = lax.axis_index("s")
    w = c * 16 + s
    _zero_buf(zbuf, K)
    _zero_acc_slice(zbuf, acc, s)
    for half in range(2):
        base = w * PH + half * PH2
        pltpu.sync_copy(pprev.at[0, pl.ds(base, PH2)], ca)
        pltpu.sync_copy(pprev.at[1, pl.ds(base, PH2)], cb)
        pltpu.sync_copy(qprev.at[pl.ds(base, PH2)], cq)
        pltpu.sync_copy(qh.at[pl.ds(base, PH2)], ch)
        pltpu.sync_copy(d2.at[pl.ds(base, PH2)], cd)
        def comb(r, carry):
            for cc in (0, 16, 32):
                sl = pl.ds(cc, 16)
                v = ch[r, sl] + cd[r, sl] * (ca[r, sl] + cb[r, sl] + cq[r, sl])
                ca[r, sl] = v
            return carry
        lax.fori_loop(0, PH2, comb, 0)
        pltpu.sync_copy(ca, q_out.at[pl.ds(base, PH2)])
    plsc.subcore_barrier()
    pltpu.core_barrier(bsem, core_axis_name="c")
    _gather_scatter(q_out, rows_hbm, cols_hbm, idxr, idxc, buf, acc, w)
    plsc.subcore_barrier()
    pltpu.sync_copy(acc.at[pl.ds(s * ROWS_PT, ROWS_PT)],
                    p_out.at[c, pl.ds(s * ROWS_PT, ROWS_PT)])


_BP = 2048


def _prep_body(degs, xr, w1, b1r, w2, b2r, al, li, tm, hh,
               q0_o, qh_o, d2_o, at_o, mu_o):
    i = pl.program_id(0)
    dis = lax.rsqrt(degs[...])
    a = jax.nn.sigmoid(al[...])
    t = tm[...]
    h = jnp.maximum(
        jnp.dot(xr[...], w1[...], preferred_element_type=_f32) + b1r[...], 0.0)
    ft = jnp.dot(h, w2[...], preferred_element_type=_f32) + b2r[...]
    q0_o[...] = dis * li[...]
    qh_o[...] = (t * dis) * hh[...]
    ridx = lax.broadcasted_iota(jnp.int32, (_BP, 1), 0) + i * _BP
    valid = (ridx < N).astype(_f32)
    d2c = valid * (1.0 - t) * dis * dis
    d2_o[...] = jnp.broadcast_to(d2c, (_BP, CP))
    at_o[...] = (t * a) * hh[...] + (1.0 - a) * ft
    mu_o[...] = valid * (1.0 - t) * a * dis


_prep = pl.pallas_call(
    _prep_body,
    grid=(NP // _BP,),
    in_specs=[
        pl.BlockSpec((_BP, 1), lambda i: (i, 0)),
        pl.BlockSpec((_BP, D), lambda i: (i, 0)),
        pl.BlockSpec((D, H), lambda i: (0, 0)),
        pl.BlockSpec((1, H), lambda i: (0, 0)),
        pl.BlockSpec((H, CP), lambda i: (0, 0)),
        pl.BlockSpec((1, CP), lambda i: (0, 0)),
        pl.BlockSpec((_BP, 1), lambda i: (i, 0)),
        pl.BlockSpec((_BP, CP), lambda i: (i, 0)),
        pl.BlockSpec((_BP, 1), lambda i: (i, 0)),
        pl.BlockSpec((_BP, CP), lambda i: (i, 0)),
    ],
    out_specs=[
        pl.BlockSpec((_BP, CP), lambda i: (i, 0)),
        pl.BlockSpec((_BP, CP), lambda i: (i, 0)),
        pl.BlockSpec((_BP, CP), lambda i: (i, 0)),
        pl.BlockSpec((_BP, CP), lambda i: (i, 0)),
        pl.BlockSpec((_BP, 1), lambda i: (i, 0)),
    ],
    out_shape=[
        jax.ShapeDtypeStruct((NP, CP), _f32),
        jax.ShapeDtypeStruct((NP, CP), _f32),
        jax.ShapeDtypeStruct((NP, CP), _f32),
        jax.ShapeDtypeStruct((NP, CP), _f32),
        jax.ShapeDtypeStruct((NP, 1), _f32),
    ],
)

_BF = 2000


def _final_body(p, qp, at, mu, out):
    srec = p[0] + p[1] + qp[...]
    out[...] = at[...] + mu[...] * srec


_final = pl.pallas_call(
    _final_body,
    grid=(N // _BF,),
    in_specs=[
        pl.BlockSpec((2, _BF, CP), lambda i: (0, i, 0)),
        pl.BlockSpec((_BF, CP), lambda i: (i, 0)),
        pl.BlockSpec((_BF, CP), lambda i: (i, 0)),
        pl.BlockSpec((_BF, 1), lambda i: (i, 0)),
    ],
    out_specs=pl.BlockSpec((_BF, CP), lambda i: (i, 0)),
    out_shape=jax.ShapeDtypeStruct((N, CP), _f32),
)


def kernel(x, edge_index, W1, b1, W2, b2, alpha, label_init, train_mask,
           hard_one_hot):
    pad_e = EPAD - E
    fill = jnp.full((pad_e,), N, jnp.int32)
    rows3 = jnp.concatenate([edge_index[0], fill]).reshape(NW, NCH, K)
    cols3 = jnp.concatenate([edge_index[1], fill]).reshape(NW, NCH, K)

    degp = _deg_kernel(rows3)
    degs = (degp[0] + degp[1] + 1.0).reshape(NP, 1)

    pn = NP - N
    xp = jnp.pad(x, ((0, pn), (0, 0)))
    li48 = jnp.pad(label_init, ((0, pn), (0, CP - C)))
    hh48 = jnp.pad(hard_one_hot, ((0, pn), (0, CP - C)))
    tm = jnp.pad(train_mask.astype(_f32), (0, pn)).reshape(NP, 1)
    al = jnp.pad(alpha, ((0, pn), (0, 0)))
    w2p = jnp.pad(W2, ((0, 0), (0, CP - C)))
    b1r = b1.reshape(1, H)
    b2r = jnp.pad(b2, (0, CP - C)).reshape(1, CP)

    q0, qh, d2, addt, mult = _prep(degs, xp, W1, b1r, w2p, b2r, al, li48, tm,
                                   hh48)

    p = _step_first(q0, rows3, cols3)
    qcur = q0
    for _ in range(STEPS - 1):
        p, qcur = _step_rest(p, qcur, qh, d2, rows3, cols3)

    out48 = _final(p, qcur, addt, mult)
    return out48[:, :C]


# speedup vs baseline: 12.3275x; 1.1844x over previous
import functools

import jax
import jax.numpy as jnp
from jax import lax
from jax.experimental import pallas as pl
from jax.experimental.pallas import tpu as pltpu
from jax.experimental.pallas import tpu_sc as plsc

N = 10000
E = 320000
D = 128
H = 256
C = 40
STEPS = 10

NP = 10240
CP = 48
K = 128
NW = 32
NCH = 80
EPW = NCH * K
EPAD = NW * EPW
ROWS_PT = NP // 16
PH = NP // NW
PHC = 80

_mesh = plsc.VectorSubcoreMesh(core_axis_name="c", subcore_axis_name="s")
_f32 = jnp.float32


def _zero_buf(ref, nrows):
    def body(r, carry):
        for cc in (0, 16, 32):
            ref[r, pl.ds(cc, 16)] = jnp.zeros((16,), _f32)
        return carry
    lax.fori_loop(0, nrows, body, 0)


def _zero_acc_slice(zbuf, acc, s):
    def body(k, carry):
        pltpu.sync_copy(zbuf, acc.at[pl.ds(s * ROWS_PT + k * K, K)])
        return carry
    lax.fori_loop(0, ROWS_PT // K, body, 0)


GS_GRP = 4
GS_SLOTS = 2 * GS_GRP


def _gather_scatter(qsrc, rows_hbm, cols_hbm, idxr, idxc, bufs, gsem, ssem,
                    acc, w):
    pltpu.sync_copy(rows_hbm.at[w], idxr)
    pltpu.sync_copy(cols_hbm.at[w], idxc)
    gd = [None] * NCH
    sd = [None] * NCH
    for j in range(GS_SLOTS):
        gd[j] = pltpu.async_copy(qsrc.at[idxr.at[j]], bufs.at[j],
                                 gsem.at[j])
    for g in range(NCH // GS_GRP):
        for b in range(GS_GRP):
            j = g * GS_GRP + b
            slot = j % GS_SLOTS
            gd[j].wait()
            sd[j] = pltpu.async_copy(bufs.at[slot], acc.at[idxc.at[j]],
                                     ssem.at[slot], add=True)
        for b in range(GS_GRP):
            j2 = (g + 2) * GS_GRP + b
            if j2 < NCH:
                slot = j2 % GS_SLOTS
                sd[g * GS_GRP + b].wait()
                gd[j2] = pltpu.async_copy(qsrc.at[idxr.at[j2]], bufs.at[slot],
                                          gsem.at[slot])
    for j in range(NCH - GS_SLOTS, NCH):
        sd[j].wait()


@functools.partial(
    pl.kernel,
    mesh=_mesh,
    compiler_params=pltpu.CompilerParams(use_tc_tiling_on_sc=False),
    out_type=jax.ShapeDtypeStruct((2, NP), _f32),
    scratch_types=[
        pltpu.VMEM((NCH, K), jnp.int32),
        pltpu.VMEM((K,), _f32),
        pltpu.VMEM((ROWS_PT,), _f32),
        pltpu.VMEM_SHARED((NP,), _f32),
    ],
)
def _deg_kernel(rows_hbm, out_hbm, idx_v, ones_v, z_v, acc):
    c = lax.axis_index("c")
    s = lax.axis_index("s")
    w = c * 16 + s
    def zb(i, carry):
        z_v[pl.ds(i * 16, 16)] = jnp.zeros((16,), _f32)
        return carry
    lax.fori_loop(0, ROWS_PT // 16, zb, 0)
    def ob(i, carry):
        ones_v[pl.ds(i * 16, 16)] = jnp.ones((16,), _f32)
        return carry
    lax.fori_loop(0, K // 16, ob, 0)
    pltpu.sync_copy(z_v, acc.at[pl.ds(s * ROWS_PT, ROWS_PT)])
    pltpu.sync_copy(rows_hbm.at[w], idx_v)
    plsc.subcore_barrier()
    def eb(j, carry):
        pltpu.sync_copy(ones_v, acc.at[idx_v.at[j]], add=True)
        return carry
    lax.fori_loop(0, NCH, eb, 0)
    plsc.subcore_barrier()
    pltpu.sync_copy(acc.at[pl.ds(s * ROWS_PT, ROWS_PT)],
                    out_hbm.at[c, pl.ds(s * ROWS_PT, ROWS_PT)])


@functools.partial(
    pl.kernel,
    mesh=_mesh,
    compiler_params=pltpu.CompilerParams(use_tc_tiling_on_sc=False),
    out_type=jax.ShapeDtypeStruct((2, NP, CP), _f32),
    scratch_types=[
        pltpu.VMEM((NCH, K), jnp.int32),
        pltpu.VMEM((NCH, K), jnp.int32),
        pltpu.VMEM((GS_SLOTS, K, CP), _f32),
        pltpu.VMEM((K, CP), _f32),
        pltpu.VMEM_SHARED((NP, CP), _f32),
        pltpu.SemaphoreType.DMA((GS_SLOTS,)),
        pltpu.SemaphoreType.DMA((GS_SLOTS,)),
    ],
)
def _step_first(q0_hbm, rows_hbm, cols_hbm, p_out, idxr, idxc, bufs, zbuf,
                acc, gsem, ssem):
    c = lax.axis_index("c")
    s = lax.axis_index("s")
    w = c * 16 + s
    _zero_buf(zbuf, K)
    _zero_acc_slice(zbuf, acc, s)
    plsc.subcore_barrier()
    _gather_scatter(q0_hbm, rows_hbm, cols_hbm, idxr, idxc, bufs, gsem, ssem,
                    acc, w)
    plsc.subcore_barrier()
    pltpu.sync_copy(acc.at[pl.ds(s * ROWS_PT, ROWS_PT)],
                    p_out.at[c, pl.ds(s * ROWS_PT, ROWS_PT)])


@functools.partial(
    pl.kernel,
    mesh=_mesh,
    compiler_params=pltpu.CompilerParams(use_tc_tiling_on_sc=False),
    out_type=(jax.ShapeDtypeStruct((2, NP, CP), _f32),
              jax.ShapeDtypeStruct((NP, CP), _f32)),
    scratch_types=[
        pltpu.VMEM((NCH, K), jnp.int32),
        pltpu.VMEM((NCH, K), jnp.int32),
        pltpu.VMEM((GS_SLOTS, K, CP), _f32),
        pltpu.VMEM((K, CP), _f32),
        pltpu.VMEM((PHC, CP), _f32),
        pltpu.VMEM((PHC, CP), _f32),
        pltpu.VMEM((PHC, CP), _f32),
        pltpu.VMEM((PHC, CP), _f32),
        pltpu.VMEM((PHC, CP), _f32),
        pltpu.VMEM_SHARED((NP, CP), _f32),
        pltpu.SemaphoreType.REGULAR,
        pltpu.SemaphoreType.DMA((GS_SLOTS,)),
        pltpu.SemaphoreType.DMA((GS_SLOTS,)),
    ],
)
def _step_rest(pprev, qprev, qh, d2, rows_hbm, cols_hbm, p_out, q_out,
               idxr, idxc, bufs, zbuf, ca, cb, cq, ch, cd, acc, bsem,
               gsem, ssem):
    c = lax.axis_index("c")
    s = lax.axis_index("s")
    w = c * 16 + s
    _zero_buf(zbuf, K)
    _zero_acc_slice(zbuf, acc, s)
    for part in range(PH // PHC):
        base = w * PH + part * PHC
        pltpu.sync_copy(pprev.at[0, pl.ds(base, PHC)], ca)
        pltpu.sync_copy(pprev.at[1, pl.ds(base, PHC)], cb)
        pltpu.sync_copy(qprev.at[pl.ds(base, PHC)], cq)
        pltpu.sync_copy(qh.at[pl.ds(base, PHC)], ch)
        pltpu.sync_copy(d2.at[pl.ds(base, PHC)], cd)
        def comb(r, carry):
            for cc in (0, 16, 32):
                sl = pl.ds(cc, 16)
                v = ch[r, sl] + cd[r, sl] * (ca[r, sl] + cb[r, sl] + cq[r, sl])
                ca[r, sl] = v
            return carry
        lax.fori_loop(0, PHC, comb, 0)
        pltpu.sync_copy(ca, q_out.at[pl.ds(base, PHC)])
    plsc.subcore_barrier()
    pltpu.core_barrier(bsem, core_axis_name="c")
    _gather_scatter(q_out, rows_hbm, cols_hbm, idxr, idxc, bufs, gsem, ssem,
                    acc, w)
    plsc.subcore_barrier()
    pltpu.sync_copy(acc.at[pl.ds(s * ROWS_PT, ROWS_PT)],
                    p_out.at[c, pl.ds(s * ROWS_PT, ROWS_PT)])


_BP = 2048


def _prep_body(degs, xr, w1, b1r, w2, b2r, al, li, tm, hh,
               q0_o, qh_o, d2_o, at_o, mu_o):
    i = pl.program_id(0)
    dis = lax.rsqrt(degs[...])
    a = jax.nn.sigmoid(al[...])
    t = tm[...]
    h = jnp.maximum(
        jnp.dot(xr[...], w1[...], preferred_element_type=_f32) + b1r[...], 0.0)
    ft = jnp.dot(h, w2[...], preferred_element_type=_f32) + b2r[...]
    q0_o[...] = dis * li[...]
    qh_o[...] = (t * dis) * hh[...]
    ridx = lax.broadcasted_iota(jnp.int32, (_BP, 1), 0) + i * _BP
    valid = (ridx < N).astype(_f32)
    d2c = valid * (1.0 - t) * dis * dis
    d2_o[...] = jnp.broadcast_to(d2c, (_BP, CP))
    at_o[...] = (t * a) * hh[...] + (1.0 - a) * ft
    mu_o[...] = valid * (1.0 - t) * a * dis


_prep = pl.pallas_call(
    _prep_body,
    grid=(NP // _BP,),
    in_specs=[
        pl.BlockSpec((_BP, 1), lambda i: (i, 0)),
        pl.BlockSpec((_BP, D), lambda i: (i, 0)),
        pl.BlockSpec((D, H), lambda i: (0, 0)),
        pl.BlockSpec((1, H), lambda i: (0, 0)),
        pl.BlockSpec((H, CP), lambda i: (0, 0)),
        pl.BlockSpec((1, CP), lambda i: (0, 0)),
        pl.BlockSpec((_BP, 1), lambda i: (i, 0)),
        pl.BlockSpec((_BP, CP), lambda i: (i, 0)),
        pl.BlockSpec((_BP, 1), lambda i: (i, 0)),
        pl.BlockSpec((_BP, CP), lambda i: (i, 0)),
    ],
    out_specs=[
        pl.BlockSpec((_BP, CP), lambda i: (i, 0)),
        pl.BlockSpec((_BP, CP), lambda i: (i, 0)),
        pl.BlockSpec((_BP, CP), lambda i: (i, 0)),
        pl.BlockSpec((_BP, CP), lambda i: (i, 0)),
        pl.BlockSpec((_BP, 1), lambda i: (i, 0)),
    ],
    out_shape=[
        jax.ShapeDtypeStruct((NP, CP), _f32),
        jax.ShapeDtypeStruct((NP, CP), _f32),
        jax.ShapeDtypeStruct((NP, CP), _f32),
        jax.ShapeDtypeStruct((NP, CP), _f32),
        jax.ShapeDtypeStruct((NP, 1), _f32),
    ],
)

_BF = 2000


def _final_body(p, qp, at, mu, out):
    srec = p[0] + p[1] + qp[...]
    out[...] = at[...] + mu[...] * srec


_final = pl.pallas_call(
    _final_body,
    grid=(N // _BF,),
    in_specs=[
        pl.BlockSpec((2, _BF, CP), lambda i: (0, i, 0)),
        pl.BlockSpec((_BF, CP), lambda i: (i, 0)),
        pl.BlockSpec((_BF, CP), lambda i: (i, 0)),
        pl.BlockSpec((_BF, 1), lambda i: (i, 0)),
    ],
    out_specs=pl.BlockSpec((_BF, CP), lambda i: (i, 0)),
    out_shape=jax.ShapeDtypeStruct((N, CP), _f32),
)


def kernel(x, edge_index, W1, b1, W2, b2, alpha, label_init, train_mask,
           hard_one_hot):
    pad_e = EPAD - E
    fill = jnp.full((pad_e,), N, jnp.int32)
    rows3 = jnp.concatenate([edge_index[0], fill]).reshape(NW, NCH, K)
    cols3 = jnp.concatenate([edge_index[1], fill]).reshape(NW, NCH, K)

    degp = _deg_kernel(rows3)
    degs = (degp[0] + degp[1] + 1.0).reshape(NP, 1)

    pn = NP - N
    xp = jnp.pad(x, ((0, pn), (0, 0)))
    li48 = jnp.pad(label_init, ((0, pn), (0, CP - C)))
    hh48 = jnp.pad(hard_one_hot, ((0, pn), (0, CP - C)))
    tm = jnp.pad(train_mask.astype(_f32), (0, pn)).reshape(NP, 1)
    al = jnp.pad(alpha, ((0, pn), (0, 0)))
    w2p = jnp.pad(W2, ((0, 0), (0, CP - C)))
    b1r = b1.reshape(1, H)
    b2r = jnp.pad(b2, (0, CP - C)).reshape(1, CP)

    q0, qh, d2, addt, mult = _prep(degs, xp, W1, b1r, w2p, b2r, al, li48, tm,
                                   hh48)

    p = _step_first(q0, rows3, cols3)
    qcur = q0
    for _ in range(STEPS - 1):
        p, qcur = _step_rest(p, qcur, qh, d2, rows3, cols3)

    out48 = _final(p, qcur, addt, mult)
    return out48[:, :C]


# speedup vs baseline: 14.8164x; 1.2019x over previous
import functools

import jax
import jax.numpy as jnp
from jax import lax
from jax.experimental import pallas as pl
from jax.experimental.pallas import tpu as pltpu
from jax.experimental.pallas import tpu_sc as plsc

N = 10000
E = 320000
D = 128
H = 256
C = 40
STEPS = 10

NP = 10240
CB = 64
K = 128
NW = 32
NCH = 80
EPW = NCH * K
EPAD = NW * EPW
ROWS_PT = NP // 16

_mesh = plsc.VectorSubcoreMesh(core_axis_name="c", subcore_axis_name="s")
_f32 = jnp.float32
_bf16 = jnp.bfloat16

GS_GRP = 4
GS_SLOTS = 2 * GS_GRP


def _gather_scatter(qsrc, rows_hbm, cols_hbm, idxr, idxc, bufs, gsem, ssem,
                    acc, w):
    pltpu.sync_copy(rows_hbm.at[w], idxr)
    pltpu.sync_copy(cols_hbm.at[w], idxc)
    gd = [None] * NCH
    sd = [None] * NCH
    for j in range(GS_SLOTS):
        gd[j] = pltpu.async_copy(qsrc.at[idxr.at[j]], bufs.at[j], gsem.at[j])
    for g in range(NCH // GS_GRP):
        for b in range(GS_GRP):
            j = g * GS_GRP + b
            slot = j % GS_SLOTS
            gd[j].wait()
            sd[j] = pltpu.async_copy(bufs.at[slot], acc.at[idxc.at[j]],
                                     ssem.at[slot], add=True)
        for b in range(GS_GRP):
            j2 = (g + 2) * GS_GRP + b
            if j2 < NCH:
                slot = j2 % GS_SLOTS
                sd[g * GS_GRP + b].wait()
                gd[j2] = pltpu.async_copy(qsrc.at[idxr.at[j2]], bufs.at[slot],
                                          gsem.at[slot])
    for j in range(NCH - GS_SLOTS, NCH):
        sd[j].wait()


@functools.partial(
    pl.kernel,
    mesh=_mesh,
    compiler_params=pltpu.CompilerParams(use_tc_tiling_on_sc=False),
    out_type=jax.ShapeDtypeStruct((2, NP), _f32),
    scratch_types=[
        pltpu.VMEM((NCH, K), jnp.int32),
        pltpu.VMEM((K,), _f32),
        pltpu.VMEM((ROWS_PT,), _f32),
        pltpu.VMEM_SHARED((NP,), _f32),
    ],
)
def _deg_kernel(rows_hbm, out_hbm, idx_v, ones_v, z_v, acc):
    c = lax.axis_index("c")
    s = lax.axis_index("s")
    w = c * 16 + s
    def zb(i, carry):
        z_v[pl.ds(i * 16, 16)] = jnp.zeros((16,), _f32)
        return carry
    lax.fori_loop(0, ROWS_PT // 16, zb, 0)
    def ob(i, carry):
        ones_v[pl.ds(i * 16, 16)] = jnp.ones((16,), _f32)
        return carry
    lax.fori_loop(0, K // 16, ob, 0)
    pltpu.sync_copy(z_v, acc.at[pl.ds(s * ROWS_PT, ROWS_PT)])
    pltpu.sync_copy(rows_hbm.at[w], idx_v)
    plsc.subcore_barrier()
    def eb(j, carry):
        pltpu.sync_copy(ones_v, acc.at[idx_v.at[j]], add=True)
        return carry
    lax.fori_loop(0, NCH, eb, 0)
    plsc.subcore_barrier()
    pltpu.sync_copy(acc.at[pl.ds(s * ROWS_PT, ROWS_PT)],
                    out_hbm.at[c, pl.ds(s * ROWS_PT, ROWS_PT)])


@functools.partial(
    pl.kernel,
    mesh=_mesh,
    compiler_params=pltpu.CompilerParams(use_tc_tiling_on_sc=False),
    out_type=jax.ShapeDtypeStruct((2, NP, CB), _bf16),
    scratch_types=[
        pltpu.VMEM((NCH, K), jnp.int32),
        pltpu.VMEM((NCH, K), jnp.int32),
        pltpu.VMEM((GS_SLOTS, K, CB), _bf16),
        pltpu.VMEM((K, CB), _bf16),
        pltpu.VMEM_SHARED((NP, CB), _bf16),
        pltpu.SemaphoreType.DMA((GS_SLOTS,)),
        pltpu.SemaphoreType.DMA((GS_SLOTS,)),
    ],
)
def _step(q_hbm, rows_hbm, cols_hbm, p_out, idxr, idxc, bufs, zbuf,
          acc, gsem, ssem):
    c = lax.axis_index("c")
    s = lax.axis_index("s")
    w = c * 16 + s
    def zb(r, carry):
        for cc in (0, 32):
            zbuf[r, pl.ds(cc, 32)] = jnp.zeros((32,), _bf16)
        return carry
    lax.fori_loop(0, K, zb, 0)
    def za(k, carry):
        pltpu.sync_copy(zbuf, acc.at[pl.ds(s * ROWS_PT + k * K, K)])
        return carry
    lax.fori_loop(0, ROWS_PT // K, za, 0)
    plsc.subcore_barrier()
    _gather_scatter(q_hbm, rows_hbm, cols_hbm, idxr, idxc, bufs, gsem, ssem,
                    acc, w)
    plsc.subcore_barrier()
    pltpu.sync_copy(acc.at[pl.ds(s * ROWS_PT, ROWS_PT)],
                    p_out.at[c, pl.ds(s * ROWS_PT, ROWS_PT)])


_BP = 2048


def _prep_body(degs, xr, w1, b1r, w2, b2r, al, li, tm, hh,
               q0_o, qh_o, d2_o, at_o, mu_o):
    i = pl.program_id(0)
    dis = lax.rsqrt(degs[...])
    a = jax.nn.sigmoid(al[...])
    t = tm[...]
    h = jnp.maximum(
        jnp.dot(xr[...], w1[...], preferred_element_type=_f32) + b1r[...], 0.0)
    ft = jnp.dot(h, w2[...], preferred_element_type=_f32) + b2r[...]
    q0_o[...] = (dis * li[...]).astype(_bf16)
    qh_o[...] = (t * dis) * hh[...]
    ridx = lax.broadcasted_iota(jnp.int32, (_BP, 1), 0) + i * _BP
    valid = (ridx < N).astype(_f32)
    d2c = valid * (1.0 - t) * dis * dis
    d2_o[...] = jnp.broadcast_to(d2c, (_BP, CB))
    at_o[...] = (t * a) * hh[...] + (1.0 - a) * ft
    mu_o[...] = valid * (1.0 - t) * a * dis


_prep = pl.pallas_call(
    _prep_body,
    grid=(NP // _BP,),
    in_specs=[
        pl.BlockSpec((_BP, 1), lambda i: (i, 0)),
        pl.BlockSpec((_BP, D), lambda i: (i, 0)),
        pl.BlockSpec((D, H), lambda i: (0, 0)),
        pl.BlockSpec((1, H), lambda i: (0, 0)),
        pl.BlockSpec((H, CB), lambda i: (0, 0)),
        pl.BlockSpec((1, CB), lambda i: (0, 0)),
        pl.BlockSpec((_BP, 1), lambda i: (i, 0)),
        pl.BlockSpec((_BP, CB), lambda i: (i, 0)),
        pl.BlockSpec((_BP, 1), lambda i: (i, 0)),
        pl.BlockSpec((_BP, CB), lambda i: (i, 0)),
    ],
    out_specs=[
        pl.BlockSpec((_BP, CB), lambda i: (i, 0)),
        pl.BlockSpec((_BP, CB), lambda i: (i, 0)),
        pl.BlockSpec((_BP, CB), lambda i: (i, 0)),
        pl.BlockSpec((_BP, CB), lambda i: (i, 0)),
        pl.BlockSpec((_BP, 1), lambda i: (i, 0)),
    ],
    out_shape=[
        jax.ShapeDtypeStruct((NP, CB), _bf16),
        jax.ShapeDtypeStruct((NP, CB), _f32),
        jax.ShapeDtypeStruct((NP, CB), _f32),
        jax.ShapeDtypeStruct((NP, CB), _f32),
        jax.ShapeDtypeStruct((NP, 1), _f32),
    ],
)


def _comb_body(p, qp, qh, d2, out):
    srec = (p[0].astype(_f32) + p[1].astype(_f32) + qp[...].astype(_f32))
    out[...] = (qh[...] + d2[...] * srec).astype(_bf16)


_comb = pl.pallas_call(
    _comb_body,
    grid=(NP // _BP,),
    in_specs=[
        pl.BlockSpec((2, _BP, CB), lambda i: (0, i, 0)),
        pl.BlockSpec((_BP, CB), lambda i: (i, 0)),
        pl.BlockSpec((_BP, CB), lambda i: (i, 0)),
        pl.BlockSpec((_BP, CB), lambda i: (i, 0)),
    ],
    out_specs=pl.BlockSpec((_BP, CB), lambda i: (i, 0)),
    out_shape=jax.ShapeDtypeStruct((NP, CB), _bf16),
)

_BF = 2000


def _final_body(p, qp, at, mu, out):
    srec = (p[0].astype(_f32) + p[1].astype(_f32) + qp[...].astype(_f32))
    out[...] = at[...] + mu[...] * srec


_final = pl.pallas_call(
    _final_body,
    grid=(N // _BF,),
    in_specs=[
        pl.BlockSpec((2, _BF, CB), lambda i: (0, i, 0)),
        pl.BlockSpec((_BF, CB), lambda i: (i, 0)),
        pl.BlockSpec((_BF, CB), lambda i: (i, 0)),
        pl.BlockSpec((_BF, 1), lambda i: (i, 0)),
    ],
    out_specs=pl.BlockSpec((_BF, CB), lambda i: (i, 0)),
    out_shape=jax.ShapeDtypeStruct((N, CB), _f32),
)


def kernel(x, edge_index, W1, b1, W2, b2, alpha, label_init, train_mask,
           hard_one_hot):
    pad_e = EPAD - E
    fill = jnp.full((pad_e,), N, jnp.int32)
    rows3 = jnp.concatenate([edge_index[0], fill]).reshape(NW, NCH, K)
    cols3 = jnp.concatenate([edge_index[1], fill]).reshape(NW, NCH, K)

    degp = _deg_kernel(rows3)
    degs = (degp[0] + degp[1] + 1.0).reshape(NP, 1)

    pn = NP - N
    xp = jnp.pad(x, ((0, pn), (0, 0)))
    li64 = jnp.pad(label_init, ((0, pn), (0, CB - C)))
    hh64 = jnp.pad(hard_one_hot, ((0, pn), (0, CB - C)))
    tm = jnp.pad(train_mask.astype(_f32), (0, pn)).reshape(NP, 1)
    al = jnp.pad(alpha, ((0, pn), (0, 0)))
    w2p = jnp.pad(W2, ((0, 0), (0, CB - C)))
    b1r = b1.reshape(1, H)
    b2r = jnp.pad(b2, (0, CB - C)).reshape(1, CB)

    q0, qh, d2, addt, mult = _prep(degs, xp, W1, b1r, w2p, b2r, al, li64, tm,
                                   hh64)

    qcur = q0
    p = _step(qcur, rows3, cols3)
    for _ in range(STEPS - 1):
        qnext = _comb(p, qcur, qh, d2)
        qcur = qnext
        p = _step(qcur, rows3, cols3)

    out64 = _final(p, qcur, addt, mult)
    return out64[:, :C]


# speedup vs baseline: 20.0660x; 1.3543x over previous
import functools

import jax
import jax.numpy as jnp
from jax import lax
from jax.experimental import pallas as pl
from jax.experimental.pallas import tpu as pltpu
from jax.experimental.pallas import tpu_sc as plsc

N = 10000
E = 320000
D = 128
H = 256
C = 40
STEPS = 10

NP = 10240
CB = 64
K = 128
NW = 32
NCH = 80
EPW = NCH * K
EPAD = NW * EPW
ROWS_PT = NP // 16
PH = NP // NW

_mesh = plsc.VectorSubcoreMesh(core_axis_name="c", subcore_axis_name="s")
_f32 = jnp.float32
_bf16 = jnp.bfloat16

GS_GRP = 4
GS_SLOTS = 2 * GS_GRP


def _pipeline(qsrc, idxr, idxc, bufs, gsem, ssem, acc):
    gd = [None] * NCH
    sd = [None] * NCH
    for j in range(GS_SLOTS):
        gd[j] = pltpu.async_copy(qsrc.at[idxr.at[j]], bufs.at[j], gsem.at[j])
    for g in range(NCH // GS_GRP):
        for b in range(GS_GRP):
            j = g * GS_GRP + b
            slot = j % GS_SLOTS
            gd[j].wait()
            sd[j] = pltpu.async_copy(bufs.at[slot], acc.at[idxc.at[j]],
                                     ssem.at[slot], add=True)
        for b in range(GS_GRP):
            j2 = (g + 2) * GS_GRP + b
            if j2 < NCH:
                slot = j2 % GS_SLOTS
                sd[g * GS_GRP + b].wait()
                gd[j2] = pltpu.async_copy(qsrc.at[idxr.at[j2]], bufs.at[slot],
                                          gsem.at[slot])
    for j in range(NCH - GS_SLOTS, NCH):
        sd[j].wait()


@functools.partial(
    pl.kernel,
    mesh=_mesh,
    compiler_params=pltpu.CompilerParams(use_tc_tiling_on_sc=False),
    out_type=jax.ShapeDtypeStruct((2, NP), _f32),
    scratch_types=[
        pltpu.VMEM((NCH, K), jnp.int32),
        pltpu.VMEM((K,), _f32),
        pltpu.VMEM((ROWS_PT,), _f32),
        pltpu.VMEM_SHARED((NP,), _f32),
    ],
)
def _deg_kernel(rows_hbm, out_hbm, idx_v, ones_v, z_v, acc):
    c = lax.axis_index("c")
    s = lax.axis_index("s")
    w = c * 16 + s
    def zb(i, carry):
        z_v[pl.ds(i * 16, 16)] = jnp.zeros((16,), _f32)
        return carry
    lax.fori_loop(0, ROWS_PT // 16, zb, 0)
    def ob(i, carry):
        ones_v[pl.ds(i * 16, 16)] = jnp.ones((16,), _f32)
        return carry
    lax.fori_loop(0, K // 16, ob, 0)
    pltpu.sync_copy(z_v, acc.at[pl.ds(s * ROWS_PT, ROWS_PT)])
    pltpu.sync_copy(rows_hbm.at[w], idx_v)
    plsc.subcore_barrier()
    def eb(j, carry):
        pltpu.sync_copy(ones_v, acc.at[idx_v.at[j]], add=True)
        return carry
    lax.fori_loop(0, NCH, eb, 0)
    plsc.subcore_barrier()
    pltpu.sync_copy(acc.at[pl.ds(s * ROWS_PT, ROWS_PT)],
                    out_hbm.at[c, pl.ds(s * ROWS_PT, ROWS_PT)])


@functools.partial(
    pl.kernel,
    mesh=_mesh,
    compiler_params=pltpu.CompilerParams(use_tc_tiling_on_sc=False),
    out_type=(jax.ShapeDtypeStruct((2, NP, CB), _bf16),
              jax.ShapeDtypeStruct((NP, CB), _bf16)),
    scratch_types=[
        pltpu.VMEM((NCH, K), jnp.int32),
        pltpu.VMEM((NCH, K), jnp.int32),
        pltpu.VMEM((GS_SLOTS, K, CB), _bf16),
        pltpu.VMEM((K, CB), _bf16),
        pltpu.VMEM((PH, CB), _bf16),
        pltpu.VMEM((PH, CB), _bf16),
        pltpu.VMEM((PH, CB), _bf16),
        pltpu.VMEM((PH, CB), _bf16),
        pltpu.VMEM((PH, CB), _bf16),
        pltpu.VMEM_SHARED((NP, CB), _bf16),
        pltpu.SemaphoreType.REGULAR,
        pltpu.SemaphoreType.DMA((GS_SLOTS,)),
        pltpu.SemaphoreType.DMA((GS_SLOTS,)),
    ],
)
def _mega(q0_hbm, qh_hbm, d2_hbm, rows_hbm, cols_hbm, p_out, q_out,
          idxr, idxc, bufs, zbuf, qhb, d2b, pa, pb, qlocal, acc, bsem,
          gsem, ssem):
    c = lax.axis_index("c")
    s = lax.axis_index("s")
    w = c * 16 + s
    myrows = pl.ds(w * PH, PH)
    accslice = pl.ds(s * ROWS_PT, ROWS_PT)

    def zb(r, carry):
        for cc in (0, 32):
            zbuf[r, pl.ds(cc, 32)] = jnp.zeros((32,), _bf16)
        return carry
    lax.fori_loop(0, K, zb, 0)

    def zero_acc():
        def za(k, carry):
            pltpu.sync_copy(zbuf, acc.at[pl.ds(s * ROWS_PT + k * K, K)])
            return carry
        lax.fori_loop(0, ROWS_PT // K, za, 0)

    pltpu.sync_copy(rows_hbm.at[w], idxr)
    pltpu.sync_copy(cols_hbm.at[w], idxc)
    pltpu.sync_copy(qh_hbm.at[myrows], qhb)
    pltpu.sync_copy(d2_hbm.at[myrows], d2b)
    pltpu.sync_copy(q0_hbm.at[myrows], qlocal)
    pltpu.sync_copy(qlocal, q_out.at[myrows])
    zero_acc()
    plsc.subcore_barrier()
    pltpu.core_barrier(bsem, core_axis_name="c")

    def step(t, carry):
        @pl.when(t > 0)
        def _comb():
            pltpu.sync_copy(p_out.at[0, myrows], pa)
            pltpu.sync_copy(p_out.at[1, myrows], pb)
            def comb(r, carry2):
                for cc in (0, 32):
                    sl = pl.ds(cc, 32)
                    v = qhb[r, sl] + d2b[r, sl] * (
                        pa[r, sl] + pb[r, sl] + qlocal[r, sl])
                    qlocal[r, sl] = v
                return carry2
            lax.fori_loop(0, PH, comb, 0)
            pltpu.sync_copy(qlocal, q_out.at[myrows])
            zero_acc()
            plsc.subcore_barrier()
            pltpu.core_barrier(bsem, core_axis_name="c")

        _pipeline(q_out, idxr, idxc, bufs, gsem, ssem, acc)
        plsc.subcore_barrier()
        pltpu.sync_copy(acc.at[accslice], p_out.at[c, accslice])
        plsc.subcore_barrier()
        pltpu.core_barrier(bsem, core_axis_name="c")
        return carry

    lax.fori_loop(0, STEPS, step, 0)


_BP = 2048


def _prep_body(degs, xr, w1, b1r, w2, b2r, al, li, tm, hh,
               q0_o, qh_o, d2_o, at_o, mu_o):
    i = pl.program_id(0)
    dis = lax.rsqrt(degs[...])
    a = jax.nn.sigmoid(al[...])
    t = tm[...]
    h = jnp.maximum(
        jnp.dot(xr[...], w1[...], preferred_element_type=_f32) + b1r[...], 0.0)
    ft = jnp.dot(h, w2[...], preferred_element_type=_f32) + b2r[...]
    q0_o[...] = (dis * li[...]).astype(_bf16)
    qh_o[...] = ((t * dis) * hh[...]).astype(_bf16)
    ridx = lax.broadcasted_iota(jnp.int32, (_BP, 1), 0) + i * _BP
    valid = (ridx < N).astype(_f32)
    d2c = valid * (1.0 - t) * dis * dis
    d2_o[...] = jnp.broadcast_to(d2c, (_BP, CB)).astype(_bf16)
    at_o[...] = (t * a) * hh[...] + (1.0 - a) * ft
    mu_o[...] = valid * (1.0 - t) * a * dis


_prep = pl.pallas_call(
    _prep_body,
    grid=(NP // _BP,),
    in_specs=[
        pl.BlockSpec((_BP, 1), lambda i: (i, 0)),
        pl.BlockSpec((_BP, D), lambda i: (i, 0)),
        pl.BlockSpec((D, H), lambda i: (0, 0)),
        pl.BlockSpec((1, H), lambda i: (0, 0)),
        pl.BlockSpec((H, CB), lambda i: (0, 0)),
        pl.BlockSpec((1, CB), lambda i: (0, 0)),
        pl.BlockSpec((_BP, 1), lambda i: (i, 0)),
        pl.BlockSpec((_BP, CB), lambda i: (i, 0)),
        pl.BlockSpec((_BP, 1), lambda i: (i, 0)),
        pl.BlockSpec((_BP, CB), lambda i: (i, 0)),
    ],
    out_specs=[
        pl.BlockSpec((_BP, CB), lambda i: (i, 0)),
        pl.BlockSpec((_BP, CB), lambda i: (i, 0)),
        pl.BlockSpec((_BP, CB), lambda i: (i, 0)),
        pl.BlockSpec((_BP, CB), lambda i: (i, 0)),
        pl.BlockSpec((_BP, 1), lambda i: (i, 0)),
    ],
    out_shape=[
        jax.ShapeDtypeStruct((NP, CB), _bf16),
        jax.ShapeDtypeStruct((NP, CB), _bf16),
        jax.ShapeDtypeStruct((NP, CB), _bf16),
        jax.ShapeDtypeStruct((NP, CB), _f32),
        jax.ShapeDtypeStruct((NP, 1), _f32),
    ],
)

_BF = 2000


def _final_body(p, qp, at, mu, out):
    srec = (p[0].astype(_f32) + p[1].astype(_f32) + qp[...].astype(_f32))
    out[...] = at[...] + mu[...] * srec


_final = pl.pallas_call(
    _final_body,
    grid=(N // _BF,),
    in_specs=[
        pl.BlockSpec((2, _BF, CB), lambda i: (0, i, 0)),
        pl.BlockSpec((_BF, CB), lambda i: (i, 0)),
        pl.BlockSpec((_BF, CB), lambda i: (i, 0)),
        pl.BlockSpec((_BF, 1), lambda i: (i, 0)),
    ],
    out_specs=pl.BlockSpec((_BF, CB), lambda i: (i, 0)),
    out_shape=jax.ShapeDtypeStruct((N, CB), _f32),
)


def kernel(x, edge_index, W1, b1, W2, b2, alpha, label_init, train_mask,
           hard_one_hot):
    pad_e = EPAD - E
    fill = jnp.full((pad_e,), N, jnp.int32)
    rows3 = jnp.concatenate([edge_index[0], fill]).reshape(NW, NCH, K)
    cols3 = jnp.concatenate([edge_index[1], fill]).reshape(NW, NCH, K)

    degp = _deg_kernel(rows3)
    degs = (degp[0] + degp[1] + 1.0).reshape(NP, 1)

    pn = NP - N
    xp = jnp.pad(x, ((0, pn), (0, 0)))
    li64 = jnp.pad(label_init, ((0, pn), (0, CB - C)))
    hh64 = jnp.pad(hard_one_hot, ((0, pn), (0, CB - C)))
    tm = jnp.pad(train_mask.astype(_f32), (0, pn)).reshape(NP, 1)
    al = jnp.pad(alpha, ((0, pn), (0, 0)))
    w2p = jnp.pad(W2, ((0, 0), (0, CB - C)))
    b1r = b1.reshape(1, H)
    b2r = jnp.pad(b2, (0, CB - C)).reshape(1, CB)

    q0, qh, d2, addt, mult = _prep(degs, xp, W1, b1r, w2p, b2r, al, li64, tm,
                                   hh64)

    p, q9 = _mega(q0, qh, d2, rows3, cols3)

    out64 = _final(p, q9, addt, mult)
    return out64[:, :C]


# speedup vs baseline: 45.5682x; 2.2709x over previous
import functools

import jax
import jax.numpy as jnp
from jax import lax
from jax.experimental import pallas as pl
from jax.experimental.pallas import tpu as pltpu
from jax.experimental.pallas import tpu_sc as plsc

N = 10000
E = 320000
D = 128
H = 256
C = 40
STEPS = 10

NP = 10240
CB = 64
K = 128
NW = 32
NCH = 80
EPW = NCH * K
EPAD = NW * EPW
ROWS_PT = NP // 16
PH = NP // NW

_mesh = plsc.VectorSubcoreMesh(core_axis_name="c", subcore_axis_name="s")
_f32 = jnp.float32
_bf16 = jnp.bfloat16

GS_GRP = 2
GS_SLOTS = 2 * GS_GRP


def _pipeline(qsrc, idxr, idxc, bufs, gsem, ssem, acc):
    gd = [None] * NCH
    sd = [None] * NCH
    for j in range(GS_SLOTS):
        gd[j] = pltpu.async_copy(qsrc.at[idxr.at[j]], bufs.at[j], gsem.at[j])
    for g in range(NCH // GS_GRP):
        for b in range(GS_GRP):
            j = g * GS_GRP + b
            slot = j % GS_SLOTS
            gd[j].wait()
            sd[j] = pltpu.async_copy(bufs.at[slot], acc.at[idxc.at[j]],
                                     ssem.at[slot], add=True)
        for b in range(GS_GRP):
            j2 = (g + 2) * GS_GRP + b
            if j2 < NCH:
                slot = j2 % GS_SLOTS
                sd[g * GS_GRP + b].wait()
                gd[j2] = pltpu.async_copy(qsrc.at[idxr.at[j2]], bufs.at[slot],
                                          gsem.at[slot])
    for j in range(NCH - GS_SLOTS, NCH):
        sd[j].wait()


@functools.partial(
    pl.kernel,
    mesh=_mesh,
    compiler_params=pltpu.CompilerParams(use_tc_tiling_on_sc=False),
    out_type=jax.ShapeDtypeStruct((2, NP), _f32),
    scratch_types=[
        pltpu.VMEM((NCH, K), jnp.int32),
        pltpu.VMEM((K,), _f32),
        pltpu.VMEM((ROWS_PT,), _f32),
        pltpu.VMEM_SHARED((NP,), _f32),
    ],
)
def _deg_kernel(rows_hbm, out_hbm, idx_v, ones_v, z_v, acc):
    c = lax.axis_index("c")
    s = lax.axis_index("s")
    w = c * 16 + s
    def zb(i, carry):
        z_v[pl.ds(i * 16, 16)] = jnp.zeros((16,), _f32)
        return carry
    lax.fori_loop(0, ROWS_PT // 16, zb, 0)
    def ob(i, carry):
        ones_v[pl.ds(i * 16, 16)] = jnp.ones((16,), _f32)
        return carry
    lax.fori_loop(0, K // 16, ob, 0)
    pltpu.sync_copy(z_v, acc.at[pl.ds(s * ROWS_PT, ROWS_PT)])
    pltpu.sync_copy(rows_hbm.at[w], idx_v)
    plsc.subcore_barrier()
    def eb(j, carry):
        pltpu.sync_copy(ones_v, acc.at[idx_v.at[j]], add=True)
        return carry
    lax.fori_loop(0, NCH, eb, 0)
    plsc.subcore_barrier()
    pltpu.sync_copy(acc.at[pl.ds(s * ROWS_PT, ROWS_PT)],
                    out_hbm.at[c, pl.ds(s * ROWS_PT, ROWS_PT)])


@functools.partial(
    pl.kernel,
    mesh=_mesh,
    compiler_params=pltpu.CompilerParams(use_tc_tiling_on_sc=False),
    out_type=(jax.ShapeDtypeStruct((2, NP, CB), _bf16),
              jax.ShapeDtypeStruct((NP, CB), _bf16)),
    scratch_types=[
        pltpu.VMEM((NCH, K), jnp.int32),
        pltpu.VMEM((NCH, K), jnp.int32),
        pltpu.VMEM((GS_SLOTS, K, CB), _bf16),
        pltpu.VMEM((64, CB), _bf16),
        pltpu.VMEM((PH, CB), _bf16),
        pltpu.VMEM((PH, CB), _bf16),
        pltpu.VMEM((80, CB), _bf16),
        pltpu.VMEM((80, CB), _bf16),
        pltpu.VMEM((PH, CB), _bf16),
        pltpu.VMEM_SHARED((NP, CB), _bf16),
        pltpu.VMEM_SHARED((NP, CB), _bf16),
        pltpu.SemaphoreType.REGULAR,
        pltpu.SemaphoreType.DMA((GS_SLOTS,)),
        pltpu.SemaphoreType.DMA((GS_SLOTS,)),
    ],
)
def _mega(q0_hbm, qh_hbm, d2_hbm, rows_hbm, cols_hbm, p_out, q_out,
          idxr, idxc, bufs, zbuf, qhb, d2b, pa, pb, qlocal, acc, qrep, bsem,
          gsem, ssem):
    c = lax.axis_index("c")
    s = lax.axis_index("s")
    w = c * 16 + s
    myrows = pl.ds(w * PH, PH)
    otherrows = pl.ds(((1 - c) * 16 + s) * PH, PH)
    accslice = pl.ds(s * ROWS_PT, ROWS_PT)

    def zb(r, carry):
        for cc in (0, 32):
            zbuf[r, pl.ds(cc, 32)] = jnp.zeros((32,), _bf16)
        return carry
    lax.fori_loop(0, 64, zb, 0)

    def zero_acc():
        def za(k, carry):
            pltpu.sync_copy(zbuf, acc.at[pl.ds(s * ROWS_PT + k * 64, 64)])
            return carry
        lax.fori_loop(0, ROWS_PT // 64, za, 0)

    pltpu.sync_copy(rows_hbm.at[w], idxr)
    pltpu.sync_copy(cols_hbm.at[w], idxc)
    pltpu.sync_copy(qh_hbm.at[myrows], qhb)
    pltpu.sync_copy(d2_hbm.at[myrows], d2b)
    pltpu.sync_copy(q0_hbm.at[myrows], qlocal)
    pltpu.sync_copy(qlocal, q_out.at[myrows])
    pltpu.sync_copy(qlocal, qrep.at[myrows])
    zero_acc()
    plsc.subcore_barrier()
    pltpu.core_barrier(bsem, core_axis_name="c")
    pltpu.sync_copy(q_out.at[otherrows], qrep.at[otherrows])
    plsc.subcore_barrier()

    def step(t, carry):
        @pl.when(t > 0)
        def _comb():
            for part in range(PH // 80):
                pr = pl.ds(w * PH + part * 80, 80)
                pltpu.sync_copy(p_out.at[0, pr], pa)
                pltpu.sync_copy(p_out.at[1, pr], pb)
                def comb(r, carry2, _part=part):
                    rq = _part * 80 + r
                    for cc in (0, 32):
                        sl = pl.ds(cc, 32)
                        v = qhb[rq, sl] + d2b[rq, sl] * (
                            pa[r, sl] + pb[r, sl] + qlocal[rq, sl])
                        qlocal[rq, sl] = v
                    return carry2
                lax.fori_loop(0, 80, comb, 0)
            pltpu.sync_copy(qlocal, q_out.at[myrows])
            pltpu.sync_copy(qlocal, qrep.at[myrows])
            zero_acc()
            plsc.subcore_barrier()
            pltpu.core_barrier(bsem, core_axis_name="c")
            pltpu.sync_copy(q_out.at[otherrows], qrep.at[otherrows])
            plsc.subcore_barrier()

        _pipeline(qrep, idxr, idxc, bufs, gsem, ssem, acc)
        plsc.subcore_barrier()
        pltpu.sync_copy(acc.at[accslice], p_out.at[c, accslice])
        plsc.subcore_barrier()
        pltpu.core_barrier(bsem, core_axis_name="c")
        return carry

    lax.fori_loop(0, STEPS, step, 0)


_BP = 2048


def _prep_body(degs, xr, w1, b1r, w2, b2r, al, li, tm, hh,
               q0_o, qh_o, d2_o, at_o, mu_o):
    i = pl.program_id(0)
    dis = lax.rsqrt(degs[...])
    a = jax.nn.sigmoid(al[...])
    t = tm[...]
    h = jnp.maximum(
        jnp.dot(xr[...], w1[...], preferred_element_type=_f32) + b1r[...], 0.0)
    ft = jnp.dot(h, w2[...], preferred_element_type=_f32) + b2r[...]
    q0_o[...] = (dis * li[...]).astype(_bf16)
    qh_o[...] = ((t * dis) * hh[...]).astype(_bf16)
    ridx = lax.broadcasted_iota(jnp.int32, (_BP, 1), 0) + i * _BP
    valid = (ridx < N).astype(_f32)
    d2c = valid * (1.0 - t) * dis * dis
    d2_o[...] = jnp.broadcast_to(d2c, (_BP, CB)).astype(_bf16)
    at_o[...] = (t * a) * hh[...] + (1.0 - a) * ft
    mu_o[...] = valid * (1.0 - t) * a * dis


_prep = pl.pallas_call(
    _prep_body,
    grid=(NP // _BP,),
    in_specs=[
        pl.BlockSpec((_BP, 1), lambda i: (i, 0)),
        pl.BlockSpec((_BP, D), lambda i: (i, 0)),
        pl.BlockSpec((D, H), lambda i: (0, 0)),
        pl.BlockSpec((1, H), lambda i: (0, 0)),
        pl.BlockSpec((H, CB), lambda i: (0, 0)),
        pl.BlockSpec((1, CB), lambda i: (0, 0)),
        pl.BlockSpec((_BP, 1), lambda i: (i, 0)),
        pl.BlockSpec((_BP, CB), lambda i: (i, 0)),
        pl.BlockSpec((_BP, 1), lambda i: (i, 0)),
        pl.BlockSpec((_BP, CB), lambda i: (i, 0)),
    ],
    out_specs=[
        pl.BlockSpec((_BP, CB), lambda i: (i, 0)),
        pl.BlockSpec((_BP, CB), lambda i: (i, 0)),
        pl.BlockSpec((_BP, CB), lambda i: (i, 0)),
        pl.BlockSpec((_BP, CB), lambda i: (i, 0)),
        pl.BlockSpec((_BP, 1), lambda i: (i, 0)),
    ],
    out_shape=[
        jax.ShapeDtypeStruct((NP, CB), _bf16),
        jax.ShapeDtypeStruct((NP, CB), _bf16),
        jax.ShapeDtypeStruct((NP, CB), _bf16),
        jax.ShapeDtypeStruct((NP, CB), _f32),
        jax.ShapeDtypeStruct((NP, 1), _f32),
    ],
)

_BF = 2000


def _final_body(p, qp, at, mu, out):
    srec = (p[0].astype(_f32) + p[1].astype(_f32) + qp[...].astype(_f32))
    out[...] = at[...] + mu[...] * srec


_final = pl.pallas_call(
    _final_body,
    grid=(N // _BF,),
    in_specs=[
        pl.BlockSpec((2, _BF, CB), lambda i: (0, i, 0)),
        pl.BlockSpec((_BF, CB), lambda i: (i, 0)),
        pl.BlockSpec((_BF, CB), lambda i: (i, 0)),
        pl.BlockSpec((_BF, 1), lambda i: (i, 0)),
    ],
    out_specs=pl.BlockSpec((_BF, CB), lambda i: (i, 0)),
    out_shape=jax.ShapeDtypeStruct((N, CB), _f32),
)


def kernel(x, edge_index, W1, b1, W2, b2, alpha, label_init, train_mask,
           hard_one_hot):
    pad_e = EPAD - E
    fill = jnp.full((pad_e,), N, jnp.int32)
    rows3 = jnp.concatenate([edge_index[0], fill]).reshape(NW, NCH, K)
    cols3 = jnp.concatenate([edge_index[1], fill]).reshape(NW, NCH, K)

    degp = _deg_kernel(rows3)
    degs = (degp[0] + degp[1] + 1.0).reshape(NP, 1)

    pn = NP - N
    xp = jnp.pad(x, ((0, pn), (0, 0)))
    li64 = jnp.pad(label_init, ((0, pn), (0, CB - C)))
    hh64 = jnp.pad(hard_one_hot, ((0, pn), (0, CB - C)))
    tm = jnp.pad(train_mask.astype(_f32), (0, pn)).reshape(NP, 1)
    al = jnp.pad(alpha, ((0, pn), (0, 0)))
    w2p = jnp.pad(W2, ((0, 0), (0, CB - C)))
    b1r = b1.reshape(1, H)
    b2r = jnp.pad(b2, (0, CB - C)).reshape(1, CB)

    q0, qh, d2, addt, mult = _prep(degs, xp, W1, b1r, w2p, b2r, al, li64, tm,
                                   hh64)

    p, q9 = _mega(q0, qh, d2, rows3, cols3)

    out64 = _final(p, q9, addt, mult)
    return out64[:, :C]


# speedup vs baseline: 47.4009x; 1.0402x over previous
import functools

import jax
import jax.numpy as jnp
from jax import lax
from jax.experimental import pallas as pl
from jax.experimental.pallas import tpu as pltpu
from jax.experimental.pallas import tpu_sc as plsc

N = 10000
E = 320000
D = 128
H = 256
C = 40
STEPS = 10

NP = 10240
CB = 64
K = 128
NW = 32
NCH = 80
EPW = NCH * K
EPAD = NW * EPW
ROWS_PT = NP // 16
PH = NP // NW

_mesh = plsc.VectorSubcoreMesh(core_axis_name="c", subcore_axis_name="s")
_f32 = jnp.float32
_bf16 = jnp.bfloat16

GS_GRP = 2
GS_SLOTS = 2 * GS_GRP


def _pipeline(qsrc, idxr, idxc, bufs, gsem, ssem, acc):
    gd = [None] * NCH
    sd = [None] * NCH
    for j in range(GS_SLOTS):
        gd[j] = pltpu.async_copy(qsrc.at[idxr.at[j]], bufs.at[j], gsem.at[j])
    for g in range(NCH // GS_GRP):
        for b in range(GS_GRP):
            j = g * GS_GRP + b
            slot = j % GS_SLOTS
            gd[j].wait()
            sd[j] = pltpu.async_copy(bufs.at[slot], acc.at[idxc.at[j]],
                                     ssem.at[slot], add=True)
        for b in range(GS_GRP):
            j2 = (g + 2) * GS_GRP + b
            if j2 < NCH:
                slot = j2 % GS_SLOTS
                sd[g * GS_GRP + b].wait()
                gd[j2] = pltpu.async_copy(qsrc.at[idxr.at[j2]], bufs.at[slot],
                                          gsem.at[slot])
    for j in range(NCH - GS_SLOTS, NCH):
        sd[j].wait()


@functools.partial(
    pl.kernel,
    mesh=_mesh,
    compiler_params=pltpu.CompilerParams(use_tc_tiling_on_sc=False),
    out_type=jax.ShapeDtypeStruct((2, NP), _f32),
    scratch_types=[
        pltpu.VMEM((NCH, K), jnp.int32),
        pltpu.VMEM((K,), _f32),
        pltpu.VMEM((ROWS_PT,), _f32),
        pltpu.VMEM_SHARED((NP,), _f32),
    ],
)
def _deg_kernel(rows_hbm, out_hbm, idx_v, ones_v, z_v, acc):
    c = lax.axis_index("c")
    s = lax.axis_index("s")
    w = c * 16 + s
    def zb(i, carry):
        z_v[pl.ds(i * 16, 16)] = jnp.zeros((16,), _f32)
        return carry
    lax.fori_loop(0, ROWS_PT // 16, zb, 0)
    def ob(i, carry):
        ones_v[pl.ds(i * 16, 16)] = jnp.ones((16,), _f32)
        return carry
    lax.fori_loop(0, K // 16, ob, 0)
    pltpu.sync_copy(z_v, acc.at[pl.ds(s * ROWS_PT, ROWS_PT)])
    pltpu.sync_copy(rows_hbm.at[w], idx_v)
    plsc.subcore_barrier()
    def eb(j, carry):
        pltpu.sync_copy(ones_v, acc.at[idx_v.at[j]], add=True)
        return carry
    lax.fori_loop(0, NCH, eb, 0)
    plsc.subcore_barrier()
    pltpu.sync_copy(acc.at[pl.ds(s * ROWS_PT, ROWS_PT)],
                    out_hbm.at[c, pl.ds(s * ROWS_PT, ROWS_PT)])


@functools.partial(
    pl.kernel,
    mesh=_mesh,
    compiler_params=pltpu.CompilerParams(use_tc_tiling_on_sc=False),
    out_type=(jax.ShapeDtypeStruct((2, NP, CB), _bf16),
              jax.ShapeDtypeStruct((NP, CB), _bf16)),
    scratch_types=[
        pltpu.VMEM((NCH, K), jnp.int32),
        pltpu.VMEM((NCH, K), jnp.int32),
        pltpu.VMEM((GS_SLOTS, K, CB), _bf16),
        pltpu.VMEM((64, CB), _bf16),
        pltpu.VMEM((PH, CB), _bf16),
        pltpu.VMEM((PH, CB), _bf16),
        pltpu.VMEM((80, CB), _bf16),
        pltpu.VMEM((80, CB), _bf16),
        pltpu.VMEM((PH, CB), _bf16),
        pltpu.VMEM_SHARED((NP, CB), _bf16),
        pltpu.VMEM_SHARED((NP, CB), _bf16),
        pltpu.SemaphoreType.REGULAR,
        pltpu.SemaphoreType.DMA((GS_SLOTS,)),
        pltpu.SemaphoreType.DMA((GS_SLOTS,)),
    ],
)
def _mega(q0_hbm, qh_hbm, d2_hbm, rows_hbm, cols_hbm, p_out, q_out,
          idxr, idxc, bufs, zbuf, qhb, d2b, pa, pb, qlocal, acc, qrep, bsem,
          gsem, ssem):
    c = lax.axis_index("c")
    s = lax.axis_index("s")
    w = c * 16 + s
    myrows = pl.ds(w * PH, PH)
    otherrows = pl.ds(((1 - c) * 16 + s) * PH, PH)
    accslice = pl.ds(s * ROWS_PT, ROWS_PT)

    def zb(r, carry):
        for cc in (0, 32):
            zbuf[r, pl.ds(cc, 32)] = jnp.zeros((32,), _bf16)
        return carry
    lax.fori_loop(0, 64, zb, 0)

    def zero_acc():
        def za(k, carry):
            pltpu.sync_copy(zbuf, acc.at[pl.ds(s * ROWS_PT + k * 64, 64)])
            return carry
        lax.fori_loop(0, ROWS_PT // 64, za, 0)

    pltpu.sync_copy(rows_hbm.at[w], idxr)
    pltpu.sync_copy(cols_hbm.at[w], idxc)
    pltpu.sync_copy(qh_hbm.at[myrows], qhb)
    pltpu.sync_copy(d2_hbm.at[myrows], d2b)
    pltpu.sync_copy(q0_hbm.at[myrows], qlocal)
    pltpu.sync_copy(qlocal, q_out.at[myrows])
    pltpu.sync_copy(qlocal, qrep.at[myrows])
    zero_acc()
    plsc.subcore_barrier()
    pltpu.core_barrier(bsem, core_axis_name="c")
    pltpu.sync_copy(q_out.at[otherrows], qrep.at[otherrows])
    plsc.subcore_barrier()

    def step(t, carry):
        @pl.when(t > 0)
        def _comb():
            for part in range(PH // 80):
                pr = pl.ds(w * PH + part * 80, 80)
                pltpu.sync_copy(acc.at[pr], pa)
                pltpu.sync_copy(p_out.at[1 - c, pr], pb)
                def comb(r, carry2, _part=part):
                    rq = _part * 80 + r
                    for cc in (0, 32):
                        sl = pl.ds(cc, 32)
                        v = qhb[rq, sl] + d2b[rq, sl] * (
                            pa[r, sl] + pb[r, sl] + qlocal[rq, sl])
                        qlocal[rq, sl] = v
                    return carry2
                lax.fori_loop(0, 80, comb, 0)
            pltpu.sync_copy(qlocal, q_out.at[myrows])
            pltpu.sync_copy(qlocal, qrep.at[myrows])
            zero_acc()
            plsc.subcore_barrier()
            pltpu.core_barrier(bsem, core_axis_name="c")
            pltpu.sync_copy(q_out.at[otherrows], qrep.at[otherrows])
            plsc.subcore_barrier()

        _pipeline(qrep, idxr, idxc, bufs, gsem, ssem, acc)
        plsc.subcore_barrier()
        pltpu.sync_copy(acc.at[accslice], p_out.at[c, accslice])
        plsc.subcore_barrier()
        pltpu.core_barrier(bsem, core_axis_name="c")
        return carry

    lax.fori_loop(0, STEPS, step, 0)


_BP = 2048


def _prep_body(degs, xr, w1, b1r, w2, b2r, al, li, tm, hh,
               q0_o, qh_o, d2_o, at_o, mu_o):
    i = pl.program_id(0)
    deg = (degs[0] + degs[1] + 1.0).reshape(_BP, 1)
    dis = lax.rsqrt(deg)
    a = jax.nn.sigmoid(al[...])
    t = tm[...]
    h = jnp.maximum(
        jnp.dot(xr[...], w1[...], preferred_element_type=_f32) + b1r[...], 0.0)
    ft = jnp.dot(h, w2[...], preferred_element_type=_f32) + b2r[...]
    q0_o[...] = (dis * li[...]).astype(_bf16)
    qh_o[...] = ((t * dis) * hh[...]).astype(_bf16)
    ridx = lax.broadcasted_iota(jnp.int32, (_BP, 1), 0) + i * _BP
    valid = (ridx < N).astype(_f32)
    d2c = valid * (1.0 - t) * dis * dis
    d2_o[...] = jnp.broadcast_to(d2c, (_BP, CB)).astype(_bf16)
    at_o[...] = (t * a) * hh[...] + (1.0 - a) * ft
    mu_o[...] = valid * (1.0 - t) * a * dis


_prep = pl.pallas_call(
    _prep_body,
    grid=(NP // _BP,),
    in_specs=[
        pl.BlockSpec((2, _BP), lambda i: (0, i)),
        pl.BlockSpec((_BP, D), lambda i: (i, 0)),
        pl.BlockSpec((D, H), lambda i: (0, 0)),
        pl.BlockSpec((1, H), lambda i: (0, 0)),
        pl.BlockSpec((H, CB), lambda i: (0, 0)),
        pl.BlockSpec((1, CB), lambda i: (0, 0)),
        pl.BlockSpec((_BP, 1), lambda i: (i, 0)),
        pl.BlockSpec((_BP, CB), lambda i: (i, 0)),
        pl.BlockSpec((_BP, 1), lambda i: (i, 0)),
        pl.BlockSpec((_BP, CB), lambda i: (i, 0)),
    ],
    out_specs=[
        pl.BlockSpec((_BP, CB), lambda i: (i, 0)),
        pl.BlockSpec((_BP, CB), lambda i: (i, 0)),
        pl.BlockSpec((_BP, CB), lambda i: (i, 0)),
        pl.BlockSpec((_BP, CB), lambda i: (i, 0)),
        pl.BlockSpec((_BP, 1), lambda i: (i, 0)),
    ],
    out_shape=[
        jax.ShapeDtypeStruct((NP, CB), _bf16),
        jax.ShapeDtypeStruct((NP, CB), _bf16),
        jax.ShapeDtypeStruct((NP, CB), _bf16),
        jax.ShapeDtypeStruct((NP, CB), _f32),
        jax.ShapeDtypeStruct((NP, 1), _f32),
    ],
)

_BF = 2000


def _final_body(p, qp, at, mu, out):
    srec = (p[0].astype(_f32) + p[1].astype(_f32) + qp[...].astype(_f32))
    out[...] = at[...] + mu[...] * srec


_final = pl.pallas_call(
    _final_body,
    grid=(N // _BF,),
    in_specs=[
        pl.BlockSpec((2, _BF, CB), lambda i: (0, i, 0)),
        pl.BlockSpec((_BF, CB), lambda i: (i, 0)),
        pl.BlockSpec((_BF, CB), lambda i: (i, 0)),
        pl.BlockSpec((_BF, 1), lambda i: (i, 0)),
    ],
    out_specs=pl.BlockSpec((_BF, CB), lambda i: (i, 0)),
    out_shape=jax.ShapeDtypeStruct((N, CB), _f32),
)


def kernel(x, edge_index, W1, b1, W2, b2, alpha, label_init, train_mask,
           hard_one_hot):
    pad_e = EPAD - E
    fill = jnp.full((pad_e,), N, jnp.int32)
    rows3 = jnp.concatenate([edge_index[0], fill]).reshape(NW, NCH, K)
    cols3 = jnp.concatenate([edge_index[1], fill]).reshape(NW, NCH, K)

    degp = _deg_kernel(rows3)

    pn = NP - N
    xp = jnp.pad(x, ((0, pn), (0, 0)))
    li64 = jnp.pad(label_init, ((0, pn), (0, CB - C)))
    hh64 = jnp.pad(hard_one_hot, ((0, pn), (0, CB - C)))
    tm = jnp.pad(train_mask.astype(_f32), (0, pn)).reshape(NP, 1)
    al = jnp.pad(alpha, ((0, pn), (0, 0)))
    w2p = jnp.pad(W2, ((0, 0), (0, CB - C)))
    b1r = b1.reshape(1, H)
    b2r = jnp.pad(b2, (0, CB - C)).reshape(1, CB)

    q0, qh, d2, addt, mult = _prep(degp, xp, W1, b1r, w2p, b2r, al, li64, tm,
                                   hh64)

    p, q9 = _mega(q0, qh, d2, rows3, cols3)

    out64 = _final(p, q9, addt, mult)
    return out64[:, :C]
